# baseline (speedup 1.0000x reference)
import jax
import jax.numpy as jnp
from jax import lax
from jax.experimental import pallas as pl
from jax.experimental.pallas import tpu as pltpu
from jax.experimental.pallas import tpu_sc as plsc

_Q = 100000
_B = 16384
_D = 128
_L = 16
_NC = 2
_NS = 16
_NW = _NC * _NS
_PER_W = _B // _NW
_CH = 128
_NCH = _PER_W // _CH
_MAIN = _PER_W - _L
_TBLK = 1000
_NBLK = _Q // _TBLK
_M = _Q - _B
_NINT = _M // _TBLK - 1


def _piece_copy(f_hbm, outf_hbm, rows_v, r0, p):
    pltpu.sync_copy(
        f_hbm.at[pl.ds(pl.multiple_of(r0 * _D, 8), p * _D)],
        rows_v.at[pl.ds(0, p * _D)])
    pltpu.sync_copy(
        rows_v.at[pl.ds(0, p * _D)],
        outf_hbm.at[pl.ds(pl.multiple_of(r0 * _D, 8), p * _D)])


def _edge_fill(f_hbm, outf_hbm, rows_v, start, length):
    off = jnp.int32(0)
    rem = length
    for p in (512, 256, 128, 64, 32, 16, 8, 4, 2, 1):
        bit = rem >= p
        r0 = start + off

        @pl.when(bit)
        def _(r0=r0, p=p):
            _piece_copy(f_hbm, outf_hbm, rows_v, r0, p)

        off = off + jnp.where(bit, p, 0)
        rem = rem - jnp.where(bit, p, 0)


def _scatter_body(bf_hbm, bl_hbm, f_hbm, idx_hbm, outl_hbm, outf_hbm,
                  idx16_v, rows_v, labs_v, labs2_v, didx_v, eidx_v,
                  sem_r, sem_l, sem_s):
    c = lax.axis_index("c")
    s = lax.axis_index("s")
    wid = s * _NC + c
    base = wid * _PER_W

    cp_rows = pltpu.async_copy(
        bf_hbm.at[pl.ds(base * _D, _PER_W * _D)], rows_v, sem_r)
    cp_labs = pltpu.async_copy(bl_hbm.at[pl.ds(base, _PER_W)], labs_v, sem_l)

    pltpu.sync_copy(idx_hbm, idx16_v)
    ivec0 = idx16_v[...]
    ivec = ivec0 + base + lax.iota(jnp.int32, 16)
    for k in range(_PER_W // _L):
        d = ivec + (k * _L)
        d = jnp.where(d >= _Q, d - _Q, d)
        didx_v[k // (_CH // _L), pl.ds((k % (_CH // _L)) * _L, _L)] = d
        if k == 0:
            eidx_v[0, :] = d
        if k == _PER_W // _L - 1:
            eidx_v[1, :] = d

    idx0 = ivec0[0]
    dst = idx0 + base
    dst = jnp.where(dst >= _Q, dst - _Q, dst)
    h = (8 - (dst & 7)) & 7
    no_wrap = dst <= _Q - _PER_W

    cp_rows.wait()
    cp_labs.wait()

    for k in range(_MAIN // _L):
        labs2_v[pl.ds(k * _L, _L)] = labs_v[pl.ds(h + k * _L, _L)]

    @pl.when(no_wrap)
    def _():
        cpf = pltpu.async_copy(
            rows_v, outf_hbm.at[pl.ds(pl.multiple_of(dst * _D, 8),
                                      _PER_W * _D)], sem_s)
        cpl = pltpu.async_copy(
            labs2_v, outl_hbm.at[pl.ds(pl.multiple_of(dst + h, 8), _MAIN)],
            sem_l)
        cpf.wait()
        cpl.wait()

    @pl.when(jnp.logical_not(no_wrap))
    def _():
        def row_copy(r, _):
            dr = dst + r
            dr = jnp.where(dr >= _Q, dr - _Q, dr)
            pltpu.sync_copy(
                rows_v.at[pl.ds(pl.multiple_of(r * _D, 8), _D)],
                outf_hbm.at[pl.ds(pl.multiple_of(dr * _D, 8), _D)])
            return _
        lax.fori_loop(0, _PER_W, row_copy, 0)
        cps = []
        for j in range(_NCH):
            cps.append(pltpu.async_copy(
                labs_v.at[pl.ds(j * _CH, _CH)], outl_hbm.at[didx_v.at[j]],
                sem_l))
        for cp in cps:
            cp.wait()

    e0l = pltpu.async_copy(
        labs_v.at[pl.ds(0, _L)], outl_hbm.at[eidx_v.at[0]], sem_l)
    e1l = pltpu.async_copy(
        labs_v.at[pl.ds(_PER_W - _L, _L)], outl_hbm.at[eidx_v.at[1]], sem_l)
    e0l.wait()
    e1l.wait()

    em = idx0 + _B
    em = jnp.where(em >= _Q, em - _Q, em)
    fbi = (em + (_TBLK - 1)) // _TBLK

    @pl.when(wid == 0)
    def _():
        _edge_fill(f_hbm, outf_hbm, rows_v, em, fbi * _TBLK - em)

    @pl.when(wid == 1)
    def _():
        t0 = fbi + _NINT
        t0 = jnp.where(t0 >= _NBLK, t0 - _NBLK, t0) * _TBLK
        tlen = (_M - _NINT * _TBLK) - (fbi * _TBLK - em)
        big = tlen >= _TBLK

        @pl.when(big)
        def _():
            _piece_copy(f_hbm, outf_hbm, rows_v, t0, _TBLK // 2)
            _piece_copy(f_hbm, outf_hbm, rows_v, t0 + _TBLK // 2, _TBLK // 2)

        t1 = t0 + _TBLK
        t1 = jnp.where(t1 >= _Q, t1 - _Q, t1)
        start2 = jnp.where(big, t1, t0)
        _edge_fill(f_hbm, outf_hbm, rows_v, start2,
                   jnp.where(big, tlen - _TBLK, tlen))


_scatter_fn = pl.kernel(
    _scatter_body,
    out_type=jax.ShapeDtypeStruct((_Q * _D,), jnp.float32),
    mesh=plsc.VectorSubcoreMesh(core_axis_name="c", subcore_axis_name="s"),
    scratch_types=[
        pltpu.VMEM((_L,), jnp.int32),
        pltpu.VMEM((_PER_W * _D,), jnp.float32),
        pltpu.VMEM((_PER_W,), jnp.int32),
        pltpu.VMEM((_MAIN,), jnp.int32),
        pltpu.VMEM((_NCH, _CH), jnp.int32),
        pltpu.VMEM((2, _L), jnp.int32),
        pltpu.SemaphoreType.DMA,
        pltpu.SemaphoreType.DMA,
        pltpu.SemaphoreType.DMA,
    ],
)


def _tc_copy_body(fbi_ref, scout_hbm, f_ref, out_ref, sem):
    del scout_hbm
    fbi = fbi_ref[0]
    cps = []
    for i in range(_NINT):
        b = fbi + i
        b = jnp.where(b >= _NBLK, b - _NBLK, b)
        r0 = pl.multiple_of(b * _TBLK, 8)
        cp = pltpu.make_async_copy(
            f_ref.at[pl.ds(r0, _TBLK), :],
            out_ref.at[pl.ds(r0, _TBLK), :], sem)
        cp.start()
        cps.append(cp)
    for cp in cps:
        cp.wait()


_tc_copy = pl.pallas_call(
    _tc_copy_body,
    grid_spec=pltpu.PrefetchScalarGridSpec(
        num_scalar_prefetch=1,
        grid=(1,),
        in_specs=[pl.BlockSpec(memory_space=pl.ANY),
                  pl.BlockSpec(memory_space=pl.ANY)],
        out_specs=pl.BlockSpec(memory_space=pl.ANY),
        scratch_shapes=[pltpu.SemaphoreType.DMA],
    ),
    out_shape=jax.ShapeDtypeStruct((_Q, _D), jnp.float32),
    input_output_aliases={1: 0},
)


def kernel(batch_features, batch_labels, features, labels, index):
    idx0 = jnp.asarray(index, jnp.int32) % _Q
    idx_arr = jnp.full((_L,), idx0, jnp.int32)
    em = (idx0 + _B) % _Q
    fbi_arr = jnp.full((1,), (em + (_TBLK - 1)) // _TBLK, jnp.int32)
    l_ref = jax.new_ref(labels)
    f1 = features.reshape(-1)
    sc_out = _scatter_fn(batch_features.reshape(-1), batch_labels,
                         f1, idx_arr, l_ref)
    out_f = _tc_copy(fbi_arr, sc_out.reshape(_Q, _D), features)
    return out_f, l_ref[...]


# speedup vs baseline: 19.1328x; 19.1328x over previous
import jax
import jax.numpy as jnp
from jax import lax
from jax.experimental import pallas as pl
from jax.experimental.pallas import tpu as pltpu
from jax.experimental.pallas import tpu_sc as plsc

_Q = 100000
_B = 16384
_D = 128
_L = 16
_NC = 2
_NS = 16
_NW = _NC * _NS
_PER_W = _B // _NW
_CH = 128
_NCH = _PER_W // _CH
_MAIN = _PER_W - _L


def _scatter_body(bf_hbm, bl_hbm, idx_hbm, outf_hbm, outl_hbm,
                  idx16_v, rows_v, labs_v, labs2_v, didx_v, eidx_v,
                  sem_r, sem_l, sem_s):
    c = lax.axis_index("c")
    s = lax.axis_index("s")
    wid = s * _NC + c
    base = wid * _PER_W

    cp_rows = pltpu.async_copy(
        bf_hbm.at[pl.ds(base * _D, _PER_W * _D)], rows_v, sem_r)
    cp_labs = pltpu.async_copy(bl_hbm.at[pl.ds(base, _PER_W)], labs_v, sem_l)

    pltpu.sync_copy(idx_hbm, idx16_v)
    ivec0 = idx16_v[...]
    ivec = ivec0 + base + lax.iota(jnp.int32, 16)
    for k in range(_PER_W // _L):
        d = ivec + (k * _L)
        d = jnp.where(d >= _Q, d - _Q, d)
        didx_v[k // (_CH // _L), pl.ds((k % (_CH // _L)) * _L, _L)] = d
        if k == 0:
            eidx_v[0, :] = d
        if k == _PER_W // _L - 1:
            eidx_v[1, :] = d

    dst = ivec0[0] + base
    dst = jnp.where(dst >= _Q, dst - _Q, dst)
    h = (8 - (dst & 7)) & 7
    no_wrap = dst <= _Q - _PER_W

    cp_rows.wait()
    cp_labs.wait()

    for k in range(_MAIN // _L):
        labs2_v[pl.ds(k * _L, _L)] = labs_v[pl.ds(h + k * _L, _L)]

    @pl.when(no_wrap)
    def _():
        cpf = pltpu.async_copy(
            rows_v, outf_hbm.at[pl.ds(pl.multiple_of(dst * _D, 8),
                                      _PER_W * _D)], sem_s)
        cpl = pltpu.async_copy(
            labs2_v, outl_hbm.at[pl.ds(pl.multiple_of(dst + h, 8), _MAIN)],
            sem_l)
        cpf.wait()
        cpl.wait()

    @pl.when(jnp.logical_not(no_wrap))
    def _():
        def row_copy(r, _):
            dr = dst + r
            dr = jnp.where(dr >= _Q, dr - _Q, dr)
            pltpu.sync_copy(
                rows_v.at[pl.ds(pl.multiple_of(r * _D, 8), _D)],
                outf_hbm.at[pl.ds(pl.multiple_of(dr * _D, 8), _D)])
            return _
        lax.fori_loop(0, _PER_W, row_copy, 0)
        cps = []
        for j in range(_NCH):
            cps.append(pltpu.async_copy(
                labs_v.at[pl.ds(j * _CH, _CH)], outl_hbm.at[didx_v.at[j]],
                sem_l))
        for cp in cps:
            cp.wait()

    e0l = pltpu.async_copy(
        labs_v.at[pl.ds(0, _L)], outl_hbm.at[eidx_v.at[0]], sem_l)
    e1l = pltpu.async_copy(
        labs_v.at[pl.ds(_PER_W - _L, _L)], outl_hbm.at[eidx_v.at[1]], sem_l)
    e0l.wait()
    e1l.wait()


_scatter_fn = pl.kernel(
    _scatter_body,
    out_type=(),
    mesh=plsc.VectorSubcoreMesh(core_axis_name="c", subcore_axis_name="s"),
    scratch_types=[
        pltpu.VMEM((_L,), jnp.int32),
        pltpu.VMEM((_PER_W * _D,), jnp.float32),
        pltpu.VMEM((_PER_W,), jnp.int32),
        pltpu.VMEM((_MAIN,), jnp.int32),
        pltpu.VMEM((_NCH, _CH), jnp.int32),
        pltpu.VMEM((2, _L), jnp.int32),
        pltpu.SemaphoreType.DMA,
        pltpu.SemaphoreType.DMA,
        pltpu.SemaphoreType.DMA,
    ],
)


def kernel(batch_features, batch_labels, features, labels, index):
    idx0 = jnp.asarray(index, jnp.int32) % _Q
    idx_arr = jnp.full((_L,), idx0, jnp.int32)
    f_ref = jax.new_ref(features.reshape(-1))
    l_ref = jax.new_ref(labels)
    _scatter_fn(batch_features.reshape(-1), batch_labels, idx_arr,
                f_ref, l_ref)
    return f_ref[...].reshape(_Q, _D), l_ref[...]
